# baseline (device time: 116557 ns/iter reference)
import jax
import jax.numpy as jnp
from jax import lax
from jax.experimental import pallas as pl
from jax.experimental.pallas import tpu as pltpu

N_DEV = 8


def kernel(A, B):
    m, k = A.shape
    k2, n = B.shape
    assert k == k2
    m_out = m // N_DEV

    def body(a_ref, b_ref, out_ref, send_buf, recv_buf, send_sems, recv_sems):
        my = lax.axis_index("i")
        left = (my - 1) % N_DEV
        right = (my + 1) % N_DEV

        barrier_sem = pltpu.get_barrier_semaphore()
        for nbr in (left, right):
            pl.semaphore_signal(
                barrier_sem, inc=1,
                device_id=(nbr,), device_id_type=pl.DeviceIdType.MESH,
            )
        pl.semaphore_wait(barrier_sem, 2)

        b = b_ref[:, :]

        def partial_chunk(c):
            a_chunk = a_ref[pl.ds(c * m_out, m_out), :]
            return jax.lax.dot_general(
                a_chunk, b,
                dimension_numbers=(((1,), (0,)), ((), ())),
                preferred_element_type=jnp.float32,
            )

        for s in range(N_DEV - 1):
            c_send = (my - s - 1) % N_DEV
            if s == 0:
                send_buf[s, :, :] = partial_chunk(c_send)
            else:
                send_buf[s, :, :] = recv_buf[s - 1, :, :] + partial_chunk(c_send)
            rdma = pltpu.make_async_remote_copy(
                src_ref=send_buf.at[s],
                dst_ref=recv_buf.at[s],
                send_sem=send_sems.at[s],
                recv_sem=recv_sems.at[s],
                device_id=(right,),
                device_id_type=pl.DeviceIdType.MESH,
            )
            rdma.start()
            rdma.wait()

        out_ref[:, :] = recv_buf[N_DEV - 2, :, :] + partial_chunk(my)

    out_shape = jax.ShapeDtypeStruct((m_out, n), jnp.float32)
    return pl.pallas_call(
        body,
        out_shape=out_shape,
        in_specs=[
            pl.BlockSpec(memory_space=pltpu.VMEM),
            pl.BlockSpec(memory_space=pltpu.VMEM),
        ],
        out_specs=pl.BlockSpec(memory_space=pltpu.VMEM),
        scratch_shapes=[
            pltpu.VMEM((N_DEV - 1, m_out, n), jnp.float32),
            pltpu.VMEM((N_DEV - 1, m_out, n), jnp.float32),
            pltpu.SemaphoreType.DMA((N_DEV - 1,)),
            pltpu.SemaphoreType.DMA((N_DEV - 1,)),
        ],
        compiler_params=pltpu.CompilerParams(collective_id=0),
    )(A, B)


# device time: 71778 ns/iter; 1.6239x vs baseline; 1.6239x over previous
import jax
import jax.numpy as jnp
from jax import lax
from jax.experimental import pallas as pl
from jax.experimental.pallas import tpu as pltpu

N_DEV = 8


def kernel(A, B):
    m, k = A.shape
    k2, n = B.shape
    assert k == k2
    m_out = m // N_DEV
    n_half = n // 2

    def body(a_ref, b_ref, out_ref,
             send_r, recv_r, send_l, recv_l,
             send_sems_r, recv_sems_r, send_sems_l, recv_sems_l):
        my = lax.axis_index("i")
        left = (my - 1) % N_DEV
        right = (my + 1) % N_DEV

        barrier_sem = pltpu.get_barrier_semaphore()
        for nbr in (left, right):
            pl.semaphore_signal(
                barrier_sem, inc=1,
                device_id=(nbr,), device_id_type=pl.DeviceIdType.MESH,
            )
        pl.semaphore_wait(barrier_sem, 2)

        def partial_half(c, half):
            a_chunk = a_ref[pl.ds(c * m_out, m_out), :]
            b_half = b_ref[:, pl.ds(half * n_half, n_half)]
            return jax.lax.dot_general(
                a_chunk, b_half,
                dimension_numbers=(((1,), (0,)), ((), ())),
                preferred_element_type=jnp.float32,
            )

        def make_rdma(s, direction):
            if direction == 0:
                return pltpu.make_async_remote_copy(
                    src_ref=send_r.at[s], dst_ref=recv_r.at[s],
                    send_sem=send_sems_r.at[s], recv_sem=recv_sems_r.at[s],
                    device_id=(right,), device_id_type=pl.DeviceIdType.MESH,
                )
            return pltpu.make_async_remote_copy(
                src_ref=send_l.at[s], dst_ref=recv_l.at[s],
                send_sem=send_sems_l.at[s], recv_sem=recv_sems_l.at[s],
                device_id=(left,), device_id_type=pl.DeviceIdType.MESH,
            )

        rdmas = []
        send_r[0, :, :] = partial_half((my - 1) % N_DEV, 0)
        rdma = make_rdma(0, 0)
        rdma.start()
        rdmas.append(rdma)
        send_l[0, :, :] = partial_half((my + 1) % N_DEV, 1)
        rdma = make_rdma(0, 1)
        rdma.start()
        rdmas.append(rdma)

        for s in range(1, N_DEV - 1):
            pr = partial_half((my - s - 1) % N_DEV, 0)
            pll = partial_half((my + s + 1) % N_DEV, 1)
            rdmas[2 * (s - 1)].wait_recv()
            send_r[s, :, :] = pr + recv_r[s - 1, :, :]
            rdma = make_rdma(s, 0)
            rdma.start()
            rdmas.append(rdma)
            rdmas[2 * (s - 1) + 1].wait_recv()
            send_l[s, :, :] = pll + recv_l[s - 1, :, :]
            rdma = make_rdma(s, 1)
            rdma.start()
            rdmas.append(rdma)

        pr = partial_half(my, 0)
        pll = partial_half(my, 1)
        rdmas[2 * (N_DEV - 2)].wait_recv()
        out_ref[:, pl.ds(0, n_half)] = pr + recv_r[N_DEV - 2, :, :]
        rdmas[2 * (N_DEV - 2) + 1].wait_recv()
        out_ref[:, pl.ds(n_half, n_half)] = pll + recv_l[N_DEV - 2, :, :]

        for rdma in rdmas:
            rdma.wait_send()

    out_shape = jax.ShapeDtypeStruct((m_out, n), jnp.float32)
    nslots = N_DEV - 1
    return pl.pallas_call(
        body,
        out_shape=out_shape,
        in_specs=[
            pl.BlockSpec(memory_space=pltpu.VMEM),
            pl.BlockSpec(memory_space=pltpu.VMEM),
        ],
        out_specs=pl.BlockSpec(memory_space=pltpu.VMEM),
        scratch_shapes=[
            pltpu.VMEM((nslots, m_out, n_half), jnp.float32),
            pltpu.VMEM((nslots, m_out, n_half), jnp.float32),
            pltpu.VMEM((nslots, m_out, n_half), jnp.float32),
            pltpu.VMEM((nslots, m_out, n_half), jnp.float32),
            pltpu.SemaphoreType.DMA((nslots,)),
            pltpu.SemaphoreType.DMA((nslots,)),
            pltpu.SemaphoreType.DMA((nslots,)),
            pltpu.SemaphoreType.DMA((nslots,)),
        ],
        compiler_params=pltpu.CompilerParams(collective_id=0),
    )(A, B)


# device time: 47122 ns/iter; 2.4735x vs baseline; 1.5232x over previous
import jax
import jax.numpy as jnp
from jax import lax
from jax.experimental import pallas as pl
from jax.experimental.pallas import tpu as pltpu

N_DEV = 8
N_SCHEMES = 3


def _coords_of(d):
    r = d % 4
    return ((r + 1) // 2) % 2, r // 2, d // 4


def _pos_of(x, y, z):
    return 4 * z + 2 * y + (x + y) % 2


def _flip(coords, axes):
    x, y, z = coords
    if 0 in axes:
        x = 1 - x
    if 1 in axes:
        y = 1 - y
    if 2 in axes:
        z = 1 - z
    return x, y, z


def kernel(A, B):
    m, k = A.shape
    k2, n = B.shape
    assert k == k2
    m_out = m // N_DEV
    n_3 = n // N_SCHEMES

    def body(a_ref, b_ref, out_ref,
             acc0, acc1, acc2,
             r0_0, r0_1, r0_2,
             r1_0, r1_1, r1_2,
             r2_0, r2_1, r2_2,
             send_sems, recv_sems):
        accs = [acc0, acc1, acc2]
        recvs = [[r0_0, r0_1, r0_2], [r1_0, r1_1, r1_2], [r2_0, r2_1, r2_2]]

        my = lax.axis_index("i")
        my_coords = _coords_of(my)
        partners = [_pos_of(*_flip(my_coords, {a})) for a in range(3)]

        def slot_chunk(j, s):
            axes = {(j + t) % 3 for t in range(3) if (s >> (2 - t)) & 1}
            return _pos_of(*_flip(my_coords, axes))

        barrier_sem = pltpu.get_barrier_semaphore()
        for a in range(3):
            pl.semaphore_signal(
                barrier_sem, inc=1,
                device_id=(partners[a],), device_id_type=pl.DeviceIdType.MESH,
            )
        pl.semaphore_wait(barrier_sem, 3)

        def fill_slot(j, s):
            c = slot_chunk(j, s)
            a_chunk = a_ref[pl.ds(c * m_out, m_out), :]
            b_third = b_ref[:, pl.ds(j * n_3, n_3)]
            accs[j][s, :, :] = jax.lax.dot_general(
                a_chunk, b_third,
                dimension_numbers=(((1,), (0,)), ((), ())),
                preferred_element_type=jnp.float32,
            )

        step_src = [(4, 4), (2, 2), (1, 1)]

        def make_rdma(j, t):
            base, cnt = step_src[t]
            axis = (j + t) % 3
            return pltpu.make_async_remote_copy(
                src_ref=accs[j].at[pl.ds(base, cnt)],
                dst_ref=recvs[t][j],
                send_sem=send_sems.at[j, t],
                recv_sem=recv_sems.at[j, t],
                device_id=(partners[axis],),
                device_id_type=pl.DeviceIdType.MESH,
            )

        rdmas = {}
        for j in range(N_SCHEMES):
            for s in range(4, 8):
                fill_slot(j, s)
            rdmas[(j, 0)] = make_rdma(j, 0)
            rdmas[(j, 0)].start()

        for s in (2, 3, 1, 0):
            for j in range(N_SCHEMES):
                fill_slot(j, s)

        for j in range(N_SCHEMES):
            rdmas[(j, 0)].wait_recv()
            accs[j][2:4, :, :] = accs[j][2:4, :, :] + recvs[0][j][2:4, :, :]
            rdmas[(j, 1)] = make_rdma(j, 1)
            rdmas[(j, 1)].start()
        for j in range(N_SCHEMES):
            accs[j][0:2, :, :] = accs[j][0:2, :, :] + recvs[0][j][0:2, :, :]

        for j in range(N_SCHEMES):
            rdmas[(j, 1)].wait_recv()
            accs[j][1, :, :] = accs[j][1, :, :] + recvs[1][j][1, :, :]
            rdmas[(j, 2)] = make_rdma(j, 2)
            rdmas[(j, 2)].start()
        for j in range(N_SCHEMES):
            accs[j][0, :, :] = accs[j][0, :, :] + recvs[1][j][0, :, :]

        for j in range(N_SCHEMES):
            rdmas[(j, 2)].wait_recv()
            out_ref[:, pl.ds(j * n_3, n_3)] = (
                accs[j][0, :, :] + recvs[2][j][0, :, :]
            )

        for rdma in rdmas.values():
            rdma.wait_send()

    out_shape = jax.ShapeDtypeStruct((m_out, n), jnp.float32)
    return pl.pallas_call(
        body,
        out_shape=out_shape,
        in_specs=[
            pl.BlockSpec(memory_space=pltpu.VMEM),
            pl.BlockSpec(memory_space=pltpu.VMEM),
        ],
        out_specs=pl.BlockSpec(memory_space=pltpu.VMEM),
        scratch_shapes=[
            pltpu.VMEM((N_DEV, m_out, n_3), jnp.float32),
            pltpu.VMEM((N_DEV, m_out, n_3), jnp.float32),
            pltpu.VMEM((N_DEV, m_out, n_3), jnp.float32),
            pltpu.VMEM((4, m_out, n_3), jnp.float32),
            pltpu.VMEM((4, m_out, n_3), jnp.float32),
            pltpu.VMEM((4, m_out, n_3), jnp.float32),
            pltpu.VMEM((2, m_out, n_3), jnp.float32),
            pltpu.VMEM((2, m_out, n_3), jnp.float32),
            pltpu.VMEM((2, m_out, n_3), jnp.float32),
            pltpu.VMEM((1, m_out, n_3), jnp.float32),
            pltpu.VMEM((1, m_out, n_3), jnp.float32),
            pltpu.VMEM((1, m_out, n_3), jnp.float32),
            pltpu.SemaphoreType.DMA((N_SCHEMES, 3)),
            pltpu.SemaphoreType.DMA((N_SCHEMES, 3)),
        ],
        compiler_params=pltpu.CompilerParams(collective_id=0),
    )(A, B)


# device time: 42855 ns/iter; 2.7198x vs baseline; 1.0996x over previous
import jax
import jax.numpy as jnp
from jax import lax
from jax.experimental import pallas as pl
from jax.experimental.pallas import tpu as pltpu

N_DEV = 8
N_SCHEMES = 3


def _coords_of(d):
    r = d % 4
    return ((r + 1) // 2) % 2, r // 2, d // 4


def _pos_of(x, y, z):
    return 4 * z + 2 * y + (x + y) % 2


def _flip(coords, axes):
    x, y, z = coords
    if 0 in axes:
        x = 1 - x
    if 1 in axes:
        y = 1 - y
    if 2 in axes:
        z = 1 - z
    return x, y, z


def kernel(A, B):
    m, k = A.shape
    k2, n = B.shape
    assert k == k2
    m_out = m // N_DEV
    n_3 = n // N_SCHEMES

    def body(a_ref, b_ref, out_ref,
             acc0, acc1, acc2,
             r0_0, r0_1, r0_2,
             r1_0, r1_1, r1_2,
             r2_0, r2_1, r2_2,
             send_sems, recv_sems):
        accs = [acc0, acc1, acc2]
        recvs = [[r0_0, r0_1, r0_2], [r1_0, r1_1, r1_2], [r2_0, r2_1, r2_2]]

        my = lax.axis_index("i")
        my_coords = _coords_of(my)
        partners = [_pos_of(*_flip(my_coords, {a})) for a in range(3)]

        def slot_chunk(j, s):
            axes = {(j + t) % 3 for t in range(3) if (s >> (2 - t)) & 1}
            return _pos_of(*_flip(my_coords, axes))

        barrier_sem = pltpu.get_barrier_semaphore()
        for a in range(3):
            pl.semaphore_signal(
                barrier_sem, inc=1,
                device_id=(partners[a],), device_id_type=pl.DeviceIdType.MESH,
            )
        pl.semaphore_wait(barrier_sem, 3)

        def fill_slot(j, s):
            c = slot_chunk(j, s)
            a_chunk = a_ref[pl.ds(c * m_out, m_out), :]
            b_third = b_ref[:, pl.ds(j * n_3, n_3)]
            accs[j][s, :, :] = jax.lax.dot_general(
                a_chunk, b_third,
                dimension_numbers=(((1,), (0,)), ((), ())),
                preferred_element_type=jnp.float32,
            )

        _R0H, _R0M, _R0C, _R1H, _R1C, _R2 = range(6)
        sub = {
            _R0H: (0, 6, 2, 2),
            _R0M: (0, 5, 1, 1),
            _R0C: (0, 4, 1, 0),
            _R1H: (1, 3, 1, 1),
            _R1C: (1, 2, 1, 0),
            _R2:  (2, 1, 1, 0),
        }

        def make_rdma(j, which):
            t, base, cnt, rbase = sub[which]
            axis = (j + t) % 3
            return pltpu.make_async_remote_copy(
                src_ref=accs[j].at[pl.ds(base, cnt)],
                dst_ref=recvs[t][j].at[pl.ds(rbase, cnt)],
                send_sem=send_sems.at[j, which],
                recv_sem=recv_sems.at[j, which],
                device_id=(partners[axis],),
                device_id_type=pl.DeviceIdType.MESH,
            )

        rdmas = {}

        def fire(j, which):
            rdmas[(j, which)] = make_rdma(j, which)
            rdmas[(j, which)].start()

        for j in range(N_SCHEMES):
            fill_slot(j, 6)
            fill_slot(j, 7)
            fire(j, _R0H)
        for j in range(N_SCHEMES):
            fill_slot(j, 5)
            fire(j, _R0M)
            fill_slot(j, 4)
            fire(j, _R0C)

        for s in (3, 2, 1, 0):
            for j in range(N_SCHEMES):
                fill_slot(j, s)

        for j in range(N_SCHEMES):
            rdmas[(j, _R0H)].wait_recv()
            accs[j][2:4, :, :] = accs[j][2:4, :, :] + recvs[0][j][2:4, :, :]
            fire(j, _R1H)
            fire(j, _R1C)

        for j in range(N_SCHEMES):
            rdmas[(j, _R0M)].wait_recv()
            accs[j][1, :, :] = accs[j][1, :, :] + recvs[0][j][1, :, :]
        for j in range(N_SCHEMES):
            rdmas[(j, _R1H)].wait_recv()
            accs[j][1, :, :] = accs[j][1, :, :] + recvs[1][j][1, :, :]
            fire(j, _R2)

        for j in range(N_SCHEMES):
            rdmas[(j, _R0C)].wait_recv()
            accs[j][0, :, :] = accs[j][0, :, :] + recvs[0][j][0, :, :]
        for j in range(N_SCHEMES):
            rdmas[(j, _R1C)].wait_recv()
            accs[j][0, :, :] = accs[j][0, :, :] + recvs[1][j][0, :, :]

        for j in range(N_SCHEMES):
            rdmas[(j, _R2)].wait_recv()
            out_ref[:, pl.ds(j * n_3, n_3)] = (
                accs[j][0, :, :] + recvs[2][j][0, :, :]
            )

        for rdma in rdmas.values():
            rdma.wait_send()

    out_shape = jax.ShapeDtypeStruct((m_out, n), jnp.float32)
    return pl.pallas_call(
        body,
        out_shape=out_shape,
        in_specs=[
            pl.BlockSpec(memory_space=pltpu.VMEM),
            pl.BlockSpec(memory_space=pltpu.VMEM),
        ],
        out_specs=pl.BlockSpec(memory_space=pltpu.VMEM),
        scratch_shapes=[
            pltpu.VMEM((N_DEV, m_out, n_3), jnp.float32),
            pltpu.VMEM((N_DEV, m_out, n_3), jnp.float32),
            pltpu.VMEM((N_DEV, m_out, n_3), jnp.float32),
            pltpu.VMEM((4, m_out, n_3), jnp.float32),
            pltpu.VMEM((4, m_out, n_3), jnp.float32),
            pltpu.VMEM((4, m_out, n_3), jnp.float32),
            pltpu.VMEM((2, m_out, n_3), jnp.float32),
            pltpu.VMEM((2, m_out, n_3), jnp.float32),
            pltpu.VMEM((2, m_out, n_3), jnp.float32),
            pltpu.VMEM((1, m_out, n_3), jnp.float32),
            pltpu.VMEM((1, m_out, n_3), jnp.float32),
            pltpu.VMEM((1, m_out, n_3), jnp.float32),
            pltpu.SemaphoreType.DMA((N_SCHEMES, 6)),
            pltpu.SemaphoreType.DMA((N_SCHEMES, 6)),
        ],
        compiler_params=pltpu.CompilerParams(collective_id=0),
    )(A, B)


# device time: 28536 ns/iter; 4.0846x vs baseline; 1.5018x over previous
import jax
import jax.numpy as jnp
from jax import lax
from jax.experimental import pallas as pl
from jax.experimental.pallas import tpu as pltpu

N_DEV = 8
N_SCHEMES = 3


def _coords_of(d):
    r = d % 4
    return ((r + 1) // 2) % 2, r // 2, d // 4


def _pos_of(x, y, z):
    return 4 * z + 2 * y + (x + y) % 2


def _flip(coords, axes):
    x, y, z = coords
    if 0 in axes:
        x = 1 - x
    if 1 in axes:
        y = 1 - y
    if 2 in axes:
        z = 1 - z
    return x, y, z


def kernel(A, B):
    m, k = A.shape
    k2, n = B.shape
    assert k == k2
    m_out = m // N_DEV
    n_3 = n // N_SCHEMES

    f32 = jnp.float32
    bf16 = jnp.bfloat16

    def body(a_ref, b_ref, out_ref,
             a16, b16,
             part0, part1, part2,
             s0_0, s0_1, s0_2,
             s1_0, s1_1, s1_2,
             s2_0, s2_1, s2_2,
             r0_0, r0_1, r0_2,
             r1_0, r1_1, r1_2,
             r2_0, r2_1, r2_2,
             send_sems, recv_sems):
        parts = [part0, part1, part2]
        sends = [[s0_0, s0_1, s0_2], [s1_0, s1_1, s1_2], [s2_0, s2_1, s2_2]]
        recvs = [[r0_0, r0_1, r0_2], [r1_0, r1_1, r1_2], [r2_0, r2_1, r2_2]]

        my = lax.axis_index("i")
        my_coords = _coords_of(my)
        partners = [_pos_of(*_flip(my_coords, {a})) for a in range(3)]

        def slot_chunk(j, s):
            axes = {(j + t) % 3 for t in range(3) if (s >> (2 - t)) & 1}
            return _pos_of(*_flip(my_coords, axes))

        barrier_sem = pltpu.get_barrier_semaphore()
        for a in range(3):
            pl.semaphore_signal(
                barrier_sem, inc=1,
                device_id=(partners[a],), device_id_type=pl.DeviceIdType.MESH,
            )
        pl.semaphore_wait(barrier_sem, 3)

        a16[:, :] = a_ref[:, :].astype(bf16)
        b16[:, :] = b_ref[:, :].astype(bf16)

        def partial(j, s):
            c = slot_chunk(j, s)
            a_chunk = a16[pl.ds(c * m_out, m_out), :]
            b_third = b16[:, pl.ds(j * n_3, n_3)]
            return jax.lax.dot_general(
                a_chunk, b_third,
                dimension_numbers=(((1,), (0,)), ((), ())),
                preferred_element_type=f32,
            )

        _R0H, _R0M, _R0C, _R1H, _R1C, _R2 = range(6)
        sub = {
            _R0H: (0, 2, 2, 2),
            _R0M: (0, 1, 1, 1),
            _R0C: (0, 0, 1, 0),
            _R1H: (1, 1, 1, 1),
            _R1C: (1, 0, 1, 0),
            _R2:  (2, 0, 1, 0),
        }

        rdmas = {}

        def fire(j, which):
            t, base, cnt, rbase = sub[which]
            rdma = pltpu.make_async_remote_copy(
                src_ref=sends[t][j].at[pl.ds(base, cnt)],
                dst_ref=recvs[t][j].at[pl.ds(rbase, cnt)],
                send_sem=send_sems.at[j, which],
                recv_sem=recv_sems.at[j, which],
                device_id=(partners[(j + t) % 3],),
                device_id_type=pl.DeviceIdType.MESH,
            )
            rdma.start()
            rdmas[(j, which)] = rdma

        for j in range(N_SCHEMES):
            sends[0][j][2, :, :] = partial(j, 6).astype(bf16)
            sends[0][j][3, :, :] = partial(j, 7).astype(bf16)
            fire(j, _R0H)
        for j in range(N_SCHEMES):
            sends[0][j][1, :, :] = partial(j, 5).astype(bf16)
            fire(j, _R0M)
            sends[0][j][0, :, :] = partial(j, 4).astype(bf16)
            fire(j, _R0C)

        for s in (3, 2, 1, 0):
            for j in range(N_SCHEMES):
                parts[j][s, :, :] = partial(j, s)

        for j in range(N_SCHEMES):
            rdmas[(j, _R0H)].wait_recv()
            sends[1][j][1, :, :] = (
                parts[j][3, :, :] + recvs[0][j][3, :, :].astype(f32)
            ).astype(bf16)
            fire(j, _R1H)
            sends[1][j][0, :, :] = (
                parts[j][2, :, :] + recvs[0][j][2, :, :].astype(f32)
            ).astype(bf16)
            fire(j, _R1C)

        for j in range(N_SCHEMES):
            rdmas[(j, _R0M)].wait_recv()
        for j in range(N_SCHEMES):
            rdmas[(j, _R1H)].wait_recv()
            sends[2][j][0, :, :] = (
                parts[j][1, :, :]
                + recvs[0][j][1, :, :].astype(f32)
                + recvs[1][j][1, :, :].astype(f32)
            ).astype(bf16)
            fire(j, _R2)

        for j in range(N_SCHEMES):
            rdmas[(j, _R0C)].wait_recv()
            rdmas[(j, _R1C)].wait_recv()
            rdmas[(j, _R2)].wait_recv()
            out_ref[:, pl.ds(j * n_3, n_3)] = (
                parts[j][0, :, :]
                + recvs[0][j][0, :, :].astype(f32)
                + recvs[1][j][0, :, :].astype(f32)
                + recvs[2][j][0, :, :].astype(f32)
            )

        for rdma in rdmas.values():
            rdma.wait_send()

    out_shape = jax.ShapeDtypeStruct((m_out, n), f32)

    def vmem(shape, dtype):
        return pltpu.VMEM(shape, dtype)

    return pl.pallas_call(
        body,
        out_shape=out_shape,
        in_specs=[
            pl.BlockSpec(memory_space=pltpu.VMEM),
            pl.BlockSpec(memory_space=pltpu.VMEM),
        ],
        out_specs=pl.BlockSpec(memory_space=pltpu.VMEM),
        scratch_shapes=[
            vmem((m, k), bf16),
            vmem((k, n), bf16),
            vmem((4, m_out, n_3), f32), vmem((4, m_out, n_3), f32),
            vmem((4, m_out, n_3), f32),
            vmem((4, m_out, n_3), bf16), vmem((4, m_out, n_3), bf16),
            vmem((4, m_out, n_3), bf16),
            vmem((2, m_out, n_3), bf16), vmem((2, m_out, n_3), bf16),
            vmem((2, m_out, n_3), bf16),
            vmem((1, m_out, n_3), bf16), vmem((1, m_out, n_3), bf16),
            vmem((1, m_out, n_3), bf16),
            vmem((4, m_out, n_3), bf16), vmem((4, m_out, n_3), bf16),
            vmem((4, m_out, n_3), bf16),
            vmem((2, m_out, n_3), bf16), vmem((2, m_out, n_3), bf16),
            vmem((2, m_out, n_3), bf16),
            vmem((1, m_out, n_3), bf16), vmem((1, m_out, n_3), bf16),
            vmem((1, m_out, n_3), bf16),
            pltpu.SemaphoreType.DMA((N_SCHEMES, 6)),
            pltpu.SemaphoreType.DMA((N_SCHEMES, 6)),
        ],
        compiler_params=pltpu.CompilerParams(collective_id=0),
    )(A, B)


# device time: 28263 ns/iter; 4.1240x vs baseline; 1.0097x over previous
import jax
import jax.numpy as jnp
from jax import lax
from jax.experimental import pallas as pl
from jax.experimental.pallas import tpu as pltpu

N_DEV = 8
N_SCHEMES = 3


def _coords_of(d):
    r = d % 4
    return ((r + 1) // 2) % 2, r // 2, d // 4


def _pos_of(x, y, z):
    return 4 * z + 2 * y + (x + y) % 2


def _flip(coords, axes):
    x, y, z = coords
    if 0 in axes:
        x = 1 - x
    if 1 in axes:
        y = 1 - y
    if 2 in axes:
        z = 1 - z
    return x, y, z


def kernel(A, B):
    m, k = A.shape
    k2, n = B.shape
    assert k == k2
    m_out = m // N_DEV
    n_3 = n // N_SCHEMES

    f32 = jnp.float32
    bf16 = jnp.bfloat16

    def body(a_ref, b_ref, out_ref,
             a16, b16,
             part0, part1, part2,
             s0_0, s0_1, s0_2,
             s1_0, s1_1, s1_2,
             s2_0, s2_1, s2_2,
             r0_0, r0_1, r0_2,
             r1_0, r1_1, r1_2,
             r2_0, r2_1, r2_2,
             send_sems, recv_sems):
        parts = [part0, part1, part2]
        sends = [[s0_0, s0_1, s0_2], [s1_0, s1_1, s1_2], [s2_0, s2_1, s2_2]]
        recvs = [[r0_0, r0_1, r0_2], [r1_0, r1_1, r1_2], [r2_0, r2_1, r2_2]]

        my = lax.axis_index("i")
        my_coords = _coords_of(my)
        partners = [_pos_of(*_flip(my_coords, {a})) for a in range(3)]

        def slot_chunk(j, s):
            axes = {(j + t) % 3 for t in range(3) if (s >> (2 - t)) & 1}
            return _pos_of(*_flip(my_coords, axes))

        barrier_sem = pltpu.get_barrier_semaphore()
        for a in range(3):
            pl.semaphore_signal(
                barrier_sem, inc=1,
                device_id=(partners[a],), device_id_type=pl.DeviceIdType.MESH,
            )
        pl.semaphore_wait(barrier_sem, 3)

        def partial(j, s, a_from_input=False):
            c = slot_chunk(j, s)
            if a_from_input:
                a_chunk = a_ref[pl.ds(c * m_out, m_out), :].astype(bf16)
            else:
                a_chunk = a16[pl.ds(c * m_out, m_out), :]
            b_third = b16[:, pl.ds(j * n_3, n_3)]
            return jax.lax.dot_general(
                a_chunk, b_third,
                dimension_numbers=(((1,), (0,)), ((), ())),
                preferred_element_type=f32,
            )

        _R0H, _R0M, _R0C, _R1H, _R1C, _R2 = range(6)
        sub = {
            _R0H: (0, 2, 2, 2),
            _R0M: (0, 1, 1, 1),
            _R0C: (0, 0, 1, 0),
            _R1H: (1, 1, 1, 1),
            _R1C: (1, 0, 1, 0),
            _R2:  (2, 0, 1, 0),
        }

        rdmas = {}

        def fire(j, which):
            t, base, cnt, rbase = sub[which]
            rdma = pltpu.make_async_remote_copy(
                src_ref=sends[t][j].at[pl.ds(base, cnt)],
                dst_ref=recvs[t][j].at[pl.ds(rbase, cnt)],
                send_sem=send_sems.at[j, which],
                recv_sem=recv_sems.at[j, which],
                device_id=(partners[(j + t) % 3],),
                device_id_type=pl.DeviceIdType.MESH,
            )
            rdma.start()
            rdmas[(j, which)] = rdma

        for j in range(N_SCHEMES):
            b16[:, pl.ds(j * n_3, n_3)] = (
                b_ref[:, pl.ds(j * n_3, n_3)].astype(bf16)
            )
            sends[0][j][2, :, :] = partial(j, 6, a_from_input=True).astype(bf16)
            sends[0][j][3, :, :] = partial(j, 7, a_from_input=True).astype(bf16)
            fire(j, _R0H)
        for j in range(N_SCHEMES):
            sends[0][j][1, :, :] = partial(j, 5, a_from_input=True).astype(bf16)
            fire(j, _R0M)
            sends[0][j][0, :, :] = partial(j, 4, a_from_input=True).astype(bf16)
            fire(j, _R0C)

        a16[:, :] = a_ref[:, :].astype(bf16)

        for s in (3, 2, 1, 0):
            for j in range(N_SCHEMES):
                parts[j][s, :, :] = partial(j, s)

        for j in range(N_SCHEMES):
            rdmas[(j, _R0H)].wait_recv()
            sends[1][j][1, :, :] = (
                parts[j][3, :, :] + recvs[0][j][3, :, :].astype(f32)
            ).astype(bf16)
            fire(j, _R1H)
            sends[1][j][0, :, :] = (
                parts[j][2, :, :] + recvs[0][j][2, :, :].astype(f32)
            ).astype(bf16)
            fire(j, _R1C)

        for j in range(N_SCHEMES):
            rdmas[(j, _R0M)].wait_recv()
        for j in range(N_SCHEMES):
            rdmas[(j, _R1H)].wait_recv()
            sends[2][j][0, :, :] = (
                parts[j][1, :, :]
                + recvs[0][j][1, :, :].astype(f32)
                + recvs[1][j][1, :, :].astype(f32)
            ).astype(bf16)
            fire(j, _R2)

        for j in range(N_SCHEMES):
            rdmas[(j, _R0C)].wait_recv()
            rdmas[(j, _R1C)].wait_recv()
            rdmas[(j, _R2)].wait_recv()
            out_ref[:, pl.ds(j * n_3, n_3)] = (
                parts[j][0, :, :]
                + recvs[0][j][0, :, :].astype(f32)
                + recvs[1][j][0, :, :].astype(f32)
                + recvs[2][j][0, :, :].astype(f32)
            )

        for rdma in rdmas.values():
            rdma.wait_send()

    out_shape = jax.ShapeDtypeStruct((m_out, n), f32)

    def vmem(shape, dtype):
        return pltpu.VMEM(shape, dtype)

    return pl.pallas_call(
        body,
        out_shape=out_shape,
        in_specs=[
            pl.BlockSpec(memory_space=pltpu.VMEM),
            pl.BlockSpec(memory_space=pltpu.VMEM),
        ],
        out_specs=pl.BlockSpec(memory_space=pltpu.VMEM),
        scratch_shapes=[
            vmem((m, k), bf16),
            vmem((k, n), bf16),
            vmem((4, m_out, n_3), f32), vmem((4, m_out, n_3), f32),
            vmem((4, m_out, n_3), f32),
            vmem((4, m_out, n_3), bf16), vmem((4, m_out, n_3), bf16),
            vmem((4, m_out, n_3), bf16),
            vmem((2, m_out, n_3), bf16), vmem((2, m_out, n_3), bf16),
            vmem((2, m_out, n_3), bf16),
            vmem((1, m_out, n_3), bf16), vmem((1, m_out, n_3), bf16),
            vmem((1, m_out, n_3), bf16),
            vmem((4, m_out, n_3), bf16), vmem((4, m_out, n_3), bf16),
            vmem((4, m_out, n_3), bf16),
            vmem((2, m_out, n_3), bf16), vmem((2, m_out, n_3), bf16),
            vmem((2, m_out, n_3), bf16),
            vmem((1, m_out, n_3), bf16), vmem((1, m_out, n_3), bf16),
            vmem((1, m_out, n_3), bf16),
            pltpu.SemaphoreType.DMA((N_SCHEMES, 6)),
            pltpu.SemaphoreType.DMA((N_SCHEMES, 6)),
        ],
        compiler_params=pltpu.CompilerParams(collective_id=0),
    )(A, B)
